# baseline (speedup 1.0000x reference)
import jax
import jax.numpy as jnp
from jax import lax
from jax.experimental import pallas as pl
from jax.experimental.pallas import tpu as pltpu
from jax.experimental.pallas import tpu_sc as plsc

MAXLEN = 200
EMBED_DIM = 128
CHUNK = 100
LANES = 16
VPR = EMBED_DIM // LANES
NG = 4
NO = 2


def _build(total_tokens):
    info = plsc.get_sparse_core_info()
    nc, ns = info.num_cores, info.num_subcores
    nw = nc * ns
    n_chunks = total_tokens // CHUNK
    cpw = n_chunks // nw
    assert n_chunks % nw == 0 and cpw % NG == 0

    mesh = plsc.VectorSubcoreMesh(core_axis_name="c", subcore_axis_name="s")

    def body(x_ref, tok_ref, pos_ref, out_ref,
             idx_v, pos_v, g0, g1, g2, g3, o0, o1,
             gs0, gs1, gs2, gs3, os0, os1):
        gbufs = (g0, g1, g2, g3)
        obufs = (o0, o1)
        gsems = (gs0, gs1, gs2, gs3)
        osems = (os0, os1)
        wid = lax.axis_index("s") * nc + lax.axis_index("c")
        c0 = wid * cpw
        row0 = c0 * CHUNK
        pltpu.sync_copy(x_ref.at[pl.ds(c0, cpw)], idx_v)
        pltpu.sync_copy(pos_ref, pos_v)

        def start_gather(j, b):
            pltpu.async_copy(tok_ref.at[idx_v.at[j]], gbufs[b], gsems[b])

        def wait_gather(b):
            pltpu.make_async_copy(
                tok_ref.at[pl.ds(0, CHUNK)], gbufs[b], gsems[b]).wait()

        def start_out(j, b):
            pltpu.async_copy(
                obufs[b], out_ref.at[pl.ds(row0 + j * CHUNK, CHUNK)], osems[b])

        def wait_out(b):
            pltpu.make_async_copy(
                obufs[b], out_ref.at[pl.ds(0, CHUNK)], osems[b]).wait()

        def add_pos(gb, ob):
            def one(i, _):
                for k in range(VPR):
                    sl = pl.ds(k * LANES, LANES)
                    obufs[ob][i, sl] = (
                        gbufs[gb][i, sl] + pos_v[(gb % 2) * CHUNK + i, sl])
                return 0
            lax.fori_loop(0, CHUNK, one, 0)

        for b in range(NG):
            start_gather(b, b)
        for b in range(NG):
            if b >= NO:
                wait_out(b % NO)
            wait_gather(b)
            add_pos(b, b % NO)
            start_out(b, b % NO)
            start_gather(b + NG, b)

        def outer(o, _):
            for b in range(NG):
                j = o * NG + b
                wait_gather(b)
                wait_out(b % NO)
                add_pos(b, b % NO)
                start_out(j, b % NO)
                start_gather(j + NG, b)
            return 0
        lax.fori_loop(1, cpw // NG - 1, outer, 0)

        for b in range(NG):
            j = cpw - NG + b
            wait_gather(b)
            wait_out(b % NO)
            add_pos(b, b % NO)
            start_out(j, b % NO)
        for b in range(NO):
            wait_out(b)

    return pl.kernel(
        body,
        out_type=jax.ShapeDtypeStruct((total_tokens, EMBED_DIM), jnp.float32),
        mesh=mesh,
        compiler_params=pltpu.CompilerParams(use_tc_tiling_on_sc=False),
        scratch_types=[
            pltpu.VMEM((cpw, CHUNK), jnp.int32),
            pltpu.VMEM((MAXLEN, EMBED_DIM), jnp.float32),
            pltpu.VMEM((CHUNK, EMBED_DIM), jnp.float32),
            pltpu.VMEM((CHUNK, EMBED_DIM), jnp.float32),
            pltpu.VMEM((CHUNK, EMBED_DIM), jnp.float32),
            pltpu.VMEM((CHUNK, EMBED_DIM), jnp.float32),
            pltpu.VMEM((CHUNK, EMBED_DIM), jnp.float32),
            pltpu.VMEM((CHUNK, EMBED_DIM), jnp.float32),
            pltpu.SemaphoreType.DMA,
            pltpu.SemaphoreType.DMA,
            pltpu.SemaphoreType.DMA,
            pltpu.SemaphoreType.DMA,
            pltpu.SemaphoreType.DMA,
            pltpu.SemaphoreType.DMA,
        ],
    )


@jax.jit
def kernel(x, token_table, pos_table):
    batch = x.shape[0]
    x2 = x.reshape(-1, CHUNK).astype(jnp.int32)
    out = _build(batch * MAXLEN)(x2, token_table, pos_table)
    return out.reshape(batch, MAXLEN, EMBED_DIM)


# speedup vs baseline: 1.0052x; 1.0052x over previous
import jax
import jax.numpy as jnp
from jax import lax
from jax.experimental import pallas as pl
from jax.experimental.pallas import tpu as pltpu
from jax.experimental.pallas import tpu_sc as plsc

MAXLEN = 200
EMBED_DIM = 128
CHUNK = 100
LANES = 16
VPR = EMBED_DIM // LANES
NB = 6
LOOK = 4


def _build(total_tokens):
    info = plsc.get_sparse_core_info()
    nc, ns = info.num_cores, info.num_subcores
    nw = nc * ns
    n_chunks = total_tokens // CHUNK
    cpw = n_chunks // nw
    tail = cpw % NB
    n_main = cpw // NB - 1
    assert n_chunks % nw == 0 and tail == cpw - (n_main + 1) * NB

    mesh = plsc.VectorSubcoreMesh(core_axis_name="c", subcore_axis_name="s")

    def body(x_ref, tok_ref, pos_ref, out_ref, *scratch):
        idx_v = scratch[0]
        pos_v = scratch[1]
        bufs = scratch[2:2 + NB]
        gsems = scratch[2 + NB:2 + 2 * NB]
        osems = scratch[2 + 2 * NB:]
        wid = lax.axis_index("s") * nc + lax.axis_index("c")
        c0 = wid * cpw
        row0 = c0 * CHUNK
        pltpu.sync_copy(x_ref.at[pl.ds(c0, cpw)], idx_v)
        pltpu.sync_copy(pos_ref, pos_v)

        def start_gather(j, b):
            pltpu.async_copy(tok_ref.at[idx_v.at[j]], bufs[b], gsems[b])

        def wait_gather(b):
            pltpu.make_async_copy(
                tok_ref.at[pl.ds(0, CHUNK)], bufs[b], gsems[b]).wait()

        def start_out(j, b):
            pltpu.async_copy(
                bufs[b], out_ref.at[pl.ds(row0 + j * CHUNK, CHUNK)], osems[b])

        def wait_out(b):
            pltpu.make_async_copy(
                bufs[b], out_ref.at[pl.ds(0, CHUNK)], osems[b]).wait()

        def add_pos(b):
            def one(i, _):
                for k in range(VPR):
                    sl = pl.ds(k * LANES, LANES)
                    plsc.addupdate(
                        bufs[b].at[i, sl], pos_v[(b % 2) * CHUNK + i, sl])
                return 0
            lax.fori_loop(0, CHUNK, one, 0)

        def step(j, b, wait_prev_out):
            wait_gather(b)
            add_pos(b)
            start_out(j, b)
            if wait_prev_out:
                wait_out((b + LOOK) % NB)
            start_gather(j + LOOK, (b + LOOK) % NB)

        for k in range(LOOK):
            start_gather(k, k)
        for j in range(NB):
            step(j, j, j >= 2)

        def outer(o, _):
            for b in range(NB):
                step(o * NB + b, b, True)
            return 0
        lax.fori_loop(1, n_main + 1, outer, 0)

        for t in range(tail):
            j = (n_main + 1) * NB + t
            b = j % NB
            wait_gather(b)
            add_pos(b)
            wait_out((b + LOOK) % NB)
            start_out(j, b)
        for t in range(2):
            wait_out((cpw - 2 + t) % NB)

    return pl.kernel(
        body,
        out_type=jax.ShapeDtypeStruct((total_tokens, EMBED_DIM), jnp.float32),
        mesh=mesh,
        compiler_params=pltpu.CompilerParams(use_tc_tiling_on_sc=False),
        scratch_types=(
            [pltpu.VMEM((cpw, CHUNK), jnp.int32),
             pltpu.VMEM((MAXLEN, EMBED_DIM), jnp.float32)]
            + [pltpu.VMEM((CHUNK, EMBED_DIM), jnp.float32)] * NB
            + [pltpu.SemaphoreType.DMA] * (2 * NB)
        ),
    )


@jax.jit
def kernel(x, token_table, pos_table):
    batch = x.shape[0]
    x2 = x.reshape(-1, CHUNK).astype(jnp.int32)
    out = _build(batch * MAXLEN)(x2, token_table, pos_table)
    return out.reshape(batch, MAXLEN, EMBED_DIM)
